# baseline (speedup 1.0000x reference)
import functools

import jax
import jax.numpy as jnp
from jax import lax
from jax.experimental import pallas as pl
from jax.experimental.pallas import tpu as pltpu
from jax.experimental.pallas import tpu_sc as plsc

_ALPHA = 4
_FPB = 16


def _fast_body(in_ref, fast_ref):
    fast_ref[...] = in_ref[...]


def _fast_copy(frames):
    C, T, H, W = frames.shape
    return pl.pallas_call(
        _fast_body,
        grid=(T // _FPB,),
        in_specs=[pl.BlockSpec((C, _FPB, H, W), lambda g: (0, g, 0, 0))],
        out_specs=pl.BlockSpec((C, _FPB, H, W), lambda g: (0, g, 0, 0)),
        out_shape=jax.ShapeDtypeStruct((C, T, H, W), frames.dtype),
    )(frames)


def _slow_gather_sc(frames3, T, n_slow):
    NR, H, W = frames3.shape
    C = NR // T
    n_tasks = C * n_slow
    mesh = plsc.VectorSubcoreMesh(core_axis_name="c", subcore_axis_name="s")
    info = plsc.get_sparse_core_info()
    nw = info.num_cores * info.num_subcores

    @functools.partial(
        pl.kernel,
        mesh=mesh,
        out_type=jax.ShapeDtypeStruct((n_tasks, H, W), frames3.dtype),
        scratch_types=[
            pltpu.VMEM((1, H, W), frames3.dtype),
        ],
    )
    def k(frames_hbm, out_hbm, buf):
        wid = lax.axis_index("s") * info.num_cores + lax.axis_index("c")

        def do_task(kk):
            c = kk // n_slow
            j = kk % n_slow
            src = c * T + _ALPHA * j + j // 5
            pltpu.sync_copy(frames_hbm.at[pl.ds(src, 1)], buf)
            pltpu.sync_copy(buf, out_hbm.at[pl.ds(kk, 1)])

        do_task(wid)

        @pl.when(wid + nw < n_tasks)
        def _():
            do_task(wid + nw)

    return k(frames3)


def kernel(frames):
    C, T, H, W = frames.shape
    n_slow = T // _ALPHA
    slow3 = _slow_gather_sc(frames.reshape(C * T, H, W), T, n_slow)
    fast = _fast_copy(frames)
    slow = slow3.reshape(C, n_slow, H, W)
    return (slow, fast)
